# baseline (speedup 1.0000x reference)
import jax
import jax.numpy as jnp
from jax import lax
from jax.experimental import pallas as pl
from jax.experimental.pallas import tpu as pltpu

L = 3
C = 5
N = 62
T = 1024
K_TOP = int(N * 0.8)


def _softmax_lane(v):
    m = jnp.max(v, axis=-1, keepdims=True)
    e = jnp.exp(v - m)
    return e / jnp.sum(e, axis=-1, keepdims=True)


def _softmax_sub(v):
    m = jnp.max(v, axis=-1, keepdims=True)
    e = jnp.exp(v - m)
    return e / jnp.sum(e.T, axis=0, keepdims=True).T


def _bf(v):
    return v.astype(jnp.bfloat16).astype(jnp.float32)


def _hw_add(p0, p1):
    i0 = lax.bitcast_convert_type(p0, jnp.int32)
    i1 = lax.bitcast_convert_type(p1, jnp.int32)
    big_is_p0 = (i0 & 0x7FFFFFFF) >= (i1 & 0x7FFFFFFF)
    ib = jnp.where(big_is_p0, i0, i1)
    ism = jnp.where(big_is_p0, i1, i0)
    eb = jnp.maximum(lax.shift_right_logical(ib, 23) & 0xFF, 1)
    es = jnp.maximum(lax.shift_right_logical(ism, 23) & 0xFF, 1)
    nclear = eb - es - 4
    mask = lax.shift_left(jnp.full_like(ism, -1), jnp.clip(nclear, 0, 23))
    ism_q = jnp.where(nclear <= 0, ism,
                      jnp.where(nclear >= 24, jnp.zeros_like(ism), ism & mask))
    return lax.bitcast_convert_type(ib, jnp.float32) + \
        lax.bitcast_convert_type(ism_q, jnp.float32)


def _mhgcn_kernel(x_ref, c1W_ref, c1b_ref, mem_ref, fcW_ref, fcb_ref,
                  gW_ref, gb_ref, out_ref, adj0_ref, adj1_ref, adj2_ref):
    adj_refs = (adj0_ref, adj1_ref, adj2_ref)
    scale = jnp.sqrt(jnp.asarray(C, dtype=jnp.float32))
    f32 = jnp.float32
    cur2 = x_ref[0]
    out_ref[0, 0:C] = cur2

    for i in range(L):
        c1W = c1W_ref[i]
        c1b = c1b_ref[i]
        mem_i = mem_ref[i]
        gW = gW_ref[i]
        gb = gb_ref[i]

        xp2 = jnp.dot(c1W, cur2,
                      preferred_element_type=f32) + c1b
        xp = xp2.reshape(C, N, T)

        xs = jnp.sum(xp, axis=2)
        l1 = lax.dot_general(xs, mem_i, (((0,), (0,)), ((), ())),
                             preferred_element_type=f32) / scale
        a1 = _softmax_sub(jnp.maximum(l1, 0.0))
        l2 = lax.dot_general(xs, xs, (((0,), (0,)), ((), ())),
                             preferred_element_type=f32) / scale
        a2 = _softmax_lane(jnp.maximum(l2, 0.0))

        fw0 = fcW_ref[i, 0:1, 0:1]
        fw1 = fcW_ref[i, 1:2, 0:1]
        fb = fcb_ref[i, 0:1, 0:1]
        z = _hw_add(_bf(a1) * _bf(fw0), _bf(a2) * _bf(fw1)) + fb
        af = _softmax_lane(z)

        jj = lax.broadcasted_iota(jnp.int32, (N, N), 1)
        rank = jnp.zeros((N, N), f32)
        for i_cand in range(N):
            col = af[:, i_cand:i_cand + 1]
            cmp = (col > af) | ((col == af) & (i_cand < jj))
            rank = rank + cmp.astype(f32)
        adj = af * (rank < K_TOP).astype(f32)
        adj_refs[i][0] = adj

        props = [lax.dot_general(adj, xp[c], (((0,), (0,)), ((), ())),
                                 preferred_element_type=f32)
                 for c in range(C)]
        prop2 = jnp.stack(props, axis=0).reshape(C, N * T)
        y2 = jnp.dot(gW, prop2, preferred_element_type=f32) + gb
        ycur2 = y2 + cur2

        base = C * (i + 1)
        out_ref[0, base:base + C] = ycur2
        cur2 = ycur2


def kernel(x, conv1_W, conv1_b, mem, fc_W, fc_b, gcn_W, gcn_b):
    B = x.shape[0]
    c1b = conv1_b.reshape(L, C, 1)
    gb = gcn_b.reshape(L, C, 1)
    fcW = fc_W.reshape(L, 2, 1)
    fcb = fc_b.reshape(L, 1, 1)

    full = lambda *shape: pl.BlockSpec(shape, lambda b: (0,) * len(shape))
    out, adj0, adj1, adj2 = pl.pallas_call(
        _mhgcn_kernel,
        grid=(B,),
        in_specs=[
            pl.BlockSpec((1, C, N * T), lambda b: (b, 0, 0)),
            full(L, C, C),
            full(L, C, 1),
            full(L, C, N),
            full(L, 2, 1),
            full(L, 1, 1),
            full(L, C, C),
            full(L, C, 1),
        ],
        out_specs=[
            pl.BlockSpec((1, 4 * C, N * T), lambda b: (b, 0, 0)),
            pl.BlockSpec((1, N, N), lambda b: (b, 0, 0)),
            pl.BlockSpec((1, N, N), lambda b: (b, 0, 0)),
            pl.BlockSpec((1, N, N), lambda b: (b, 0, 0)),
        ],
        out_shape=[
            jax.ShapeDtypeStruct((B, 4 * C, N * T), jnp.float32),
            jax.ShapeDtypeStruct((B, N, N), jnp.float32),
            jax.ShapeDtypeStruct((B, N, N), jnp.float32),
            jax.ShapeDtypeStruct((B, N, N), jnp.float32),
        ],
        compiler_params=pltpu.CompilerParams(
            dimension_semantics=("parallel",),
        ),
    )(x.reshape(B, C, N * T), conv1_W, c1b, mem, fcW, fcb, gcn_W, gb)
    return (out.reshape(B, 4 * C, N, T), adj0, adj1, adj2)


# speedup vs baseline: 1.3663x; 1.3663x over previous
import jax
import jax.numpy as jnp
from jax import lax
from jax.experimental import pallas as pl
from jax.experimental.pallas import tpu as pltpu

L = 3
C = 5
N = 62
T = 1024
K_TOP = int(N * 0.8)


def _softmax_lane(v):
    m = jnp.max(v, axis=-1, keepdims=True)
    e = jnp.exp(v - m)
    return e / jnp.sum(e, axis=-1, keepdims=True)


def _softmax_sub(v):
    m = jnp.max(v, axis=-1, keepdims=True)
    e = jnp.exp(v - m)
    return e / jnp.sum(e.T, axis=0, keepdims=True).T


def _bf(v):
    return v.astype(jnp.bfloat16).astype(jnp.float32)


def _hw_add(p0, p1):
    i0 = lax.bitcast_convert_type(p0, jnp.int32)
    i1 = lax.bitcast_convert_type(p1, jnp.int32)
    big_is_p0 = (i0 & 0x7FFFFFFF) >= (i1 & 0x7FFFFFFF)
    ib = jnp.where(big_is_p0, i0, i1)
    ism = jnp.where(big_is_p0, i1, i0)
    eb = jnp.maximum(lax.shift_right_logical(ib, 23) & 0xFF, 1)
    es = jnp.maximum(lax.shift_right_logical(ism, 23) & 0xFF, 1)
    nclear = eb - es - 4
    mask = lax.shift_left(jnp.full_like(ism, -1), jnp.clip(nclear, 0, 23))
    ism_q = jnp.where(nclear <= 0, ism,
                      jnp.where(nclear >= 24, jnp.zeros_like(ism), ism & mask))
    return lax.bitcast_convert_type(ib, jnp.float32) + \
        lax.bitcast_convert_type(ism_q, jnp.float32)


def _mhgcn_kernel(x_ref, c1W_ref, c1b_ref, mem_ref, fcW_ref, fcb_ref,
                  gW_ref, gb_ref, out_ref, adj0_ref, adj1_ref, adj2_ref):
    adj_refs = (adj0_ref, adj1_ref, adj2_ref)
    scale = jnp.sqrt(jnp.asarray(C, dtype=jnp.float32))
    f32 = jnp.float32
    cur = x_ref[0]
    out_ref[0, 0:C] = cur

    for i in range(L):
        c1W = c1W_ref[i]
        c1b = c1b_ref[i]
        mem_i = mem_ref[i]
        gW = gW_ref[i]
        gb = gb_ref[i]

        xp2 = jnp.dot(c1W, cur.reshape(C, N * T),
                      preferred_element_type=f32) + c1b
        xp = xp2.reshape(C, N, T)

        xs = jnp.sum(xp, axis=2)
        l1 = lax.dot_general(xs, mem_i, (((0,), (0,)), ((), ())),
                             preferred_element_type=f32) / scale
        a1 = _softmax_sub(jnp.maximum(l1, 0.0))
        l2 = lax.dot_general(xs, xs, (((0,), (0,)), ((), ())),
                             preferred_element_type=f32) / scale
        a2 = _softmax_lane(jnp.maximum(l2, 0.0))

        fw0 = fcW_ref[i, 0:1, 0:1]
        fw1 = fcW_ref[i, 1:2, 0:1]
        fb = fcb_ref[i, 0:1, 0:1]
        z = _hw_add(_bf(a1) * _bf(fw0), _bf(a2) * _bf(fw1)) + fb
        af = _softmax_lane(z)

        jj = lax.broadcasted_iota(jnp.int32, (N, N), 1)
        rank = jnp.zeros((N, N), f32)
        for i_cand in range(N):
            col = af[:, i_cand:i_cand + 1]
            cmp = (col > af) | ((col == af) & (i_cand < jj))
            rank = rank + cmp.astype(f32)
        adj = af * (rank < K_TOP).astype(f32)
        adj_refs[i][0] = adj

        props = [lax.dot_general(adj, xp[c], (((0,), (0,)), ((), ())),
                                 preferred_element_type=f32)
                 for c in range(C)]
        prop2 = jnp.stack(props, axis=0).reshape(C, N * T)
        y2 = jnp.dot(gW, prop2, preferred_element_type=f32) + gb
        y = y2.reshape(C, N, T) + cur

        base = C * (i + 1)
        out_ref[0, base:base + C] = y
        cur = y


def kernel(x, conv1_W, conv1_b, mem, fc_W, fc_b, gcn_W, gcn_b):
    B = x.shape[0]
    c1b = conv1_b.reshape(L, C, 1)
    gb = gcn_b.reshape(L, C, 1)
    fcW = fc_W.reshape(L, 2, 1)
    fcb = fc_b.reshape(L, 1, 1)

    full = lambda *shape: pl.BlockSpec(shape, lambda b: (0,) * len(shape))
    out, adj0, adj1, adj2 = pl.pallas_call(
        _mhgcn_kernel,
        grid=(B,),
        in_specs=[
            pl.BlockSpec((1, C, N, T), lambda b: (b, 0, 0, 0)),
            full(L, C, C),
            full(L, C, 1),
            full(L, C, N),
            full(L, 2, 1),
            full(L, 1, 1),
            full(L, C, C),
            full(L, C, 1),
        ],
        out_specs=[
            pl.BlockSpec((1, 4 * C, N, T), lambda b: (b, 0, 0, 0)),
            pl.BlockSpec((1, N, N), lambda b: (b, 0, 0)),
            pl.BlockSpec((1, N, N), lambda b: (b, 0, 0)),
            pl.BlockSpec((1, N, N), lambda b: (b, 0, 0)),
        ],
        out_shape=[
            jax.ShapeDtypeStruct((B, 4 * C, N, T), jnp.float32),
            jax.ShapeDtypeStruct((B, N, N), jnp.float32),
            jax.ShapeDtypeStruct((B, N, N), jnp.float32),
            jax.ShapeDtypeStruct((B, N, N), jnp.float32),
        ],
        compiler_params=pltpu.CompilerParams(
            dimension_semantics=("parallel",),
        ),
    )(x, conv1_W, c1b, mem, fcW, fcb, gcn_W, gb)
    return (out, adj0, adj1, adj2)
